# Initial kernel scaffold; baseline (speedup 1.0000x reference)
import jax
import jax.numpy as jnp
from jax.experimental import pallas as pl


def kernel(dist, angle, idx_kj):
    raise NotImplementedError("write your pallas kernel here")


# speedup vs baseline: 2.9523x; 2.9523x over previous
import functools

import jax
import jax.numpy as jnp
import numpy as np
from jax import lax
from jax.experimental import pallas as pl
from jax.experimental.pallas import tpu as pltpu
from jax.experimental.pallas import tpu_sc as plsc

NUM_SPHERICAL = 3
NUM_RADIAL = 6
CUTOFF = 5.0
E_EDGES = 320000
T_TRIPLETS = 960000

C_Y0 = 0.28209479177387814
C_Y1 = 0.4886025119029199
C_Y2 = 0.31539156525252005

W_TAB = 32


def _jn_np(r, n):
    if n == 0:
        return np.sin(r) / r
    if n == 1:
        return np.sin(r) / r ** 2 - np.cos(r) / r
    if n == 2:
        return (3.0 / r ** 3 - 1.0 / r) * np.sin(r) - 3.0 / r ** 2 * np.cos(r)
    if n == 3:
        return (15.0 / r ** 4 - 6.0 / r ** 2) * np.sin(r) - (15.0 / r ** 3 - 1.0 / r) * np.cos(r)
    raise NotImplementedError


def _bisect(f, a, b, iters=200):
    fa = f(a)
    for _ in range(iters):
        m = 0.5 * (a + b)
        fm = f(m)
        if fa * fm <= 0.0:
            b = m
        else:
            a = m
            fa = fm
    return 0.5 * (a + b)


def _jn_zeros(n, k):
    zerosj = np.zeros((n, k), dtype=np.float64)
    zerosj[0] = np.arange(1, k + 1) * np.pi
    points = np.arange(1, k + n) * np.pi
    racines = np.zeros(k + n - 1, dtype=np.float64)
    for i in range(1, n):
        for j in range(k + n - 1 - i):
            racines[j] = _bisect(lambda r: _jn_np(r, i), points[j], points[j + 1])
        points = racines.copy()
        zerosj[i][:k] = racines[:k]
    return zerosj


_ZEROS64 = _jn_zeros(NUM_SPHERICAL, NUM_RADIAL)
_NORM64 = np.zeros((NUM_SPHERICAL, NUM_RADIAL), dtype=np.float64)
for _o in range(NUM_SPHERICAL):
    for _i in range(NUM_RADIAL):
        _NORM64[_o, _i] = 1.0 / np.sqrt(0.5 * _jn_np(_ZEROS64[_o, _i], _o + 1) ** 2)

_zrow = np.full((1, W_TAB), 1.0, dtype=np.float64)
_nrow = np.zeros((1, W_TAB), dtype=np.float64)
_zrow[0, :18] = _ZEROS64.reshape(-1).astype(np.float32).astype(np.float64)
_nrow[0, :18] = _NORM64.reshape(-1).astype(np.float32).astype(np.float64)
_nrow[0, :6] *= C_Y0
Z_ROW = (_zrow / CUTOFF).astype(np.float32)
N_ROW = _nrow.astype(np.float32)

_TAB_ROWS = E_EDGES // 4
_TAB_BLOCK = 320

Z_TILE = np.tile(Z_ROW, (1, 4))
N_TILE = np.tile(N_ROW, (1, 4))


def _table_body(d_ref, z_ref, n_ref, o_ref):
    x = d_ref[...] * z_ref[...]
    s = jnp.sin(x)
    c = jnp.cos(x)
    inv = 1.0 / x
    j0 = s * inv
    j1 = (s * inv - c) * inv
    j2 = (s * (3.0 * inv * inv - 1.0) - 3.0 * c * inv) * inv
    col = lax.broadcasted_iota(jnp.int32, x.shape, 1) % W_TAB
    pick = jnp.where(col < 6, j0, jnp.where(col < 12, j1, j2))
    o_ref[...] = pick * n_ref[...]


def _build_table(dist_rep):
    grid = (_TAB_ROWS // _TAB_BLOCK,)
    return pl.pallas_call(
        _table_body,
        grid=grid,
        in_specs=[
            pl.BlockSpec((_TAB_BLOCK, 128), lambda i: (i, 0)),
            pl.BlockSpec((1, 128), lambda i: (0, 0)),
            pl.BlockSpec((1, 128), lambda i: (0, 0)),
        ],
        out_specs=pl.BlockSpec((_TAB_BLOCK, 128), lambda i: (i, 0)),
        out_shape=jax.ShapeDtypeStruct((_TAB_ROWS, 128), jnp.float32),
    )(dist_rep, Z_TILE, N_TILE)


def _cos_body(a_ref, o_ref):
    o_ref[...] = jnp.cos(a_ref[...])


def _build_cos(angle):
    return pl.pallas_call(
        _cos_body,
        out_shape=jax.ShapeDtypeStruct((T_TRIPLETS,), jnp.float32),
    )(angle)


NW = 32
RPW = T_TRIPLETS // NW
CB = 1200
NCHUNK = RPW // CB
GW = 80
NWIN = CB // GW
GRP = CB // 16


def _sc_body(tab_hbm, cos_hbm, idx_hbm, out_hbm, idx_v, buf, obuf, cos_v, sem):
    wid = lax.axis_index("s") * 2 + lax.axis_index("c")

    @pl.loop(0, NCHUNK)
    def _chunk(k):
        base = wid * RPW + k * CB
        pltpu.sync_copy(idx_hbm.at[pl.ds(base, CB)], idx_v)
        copies = [
            pltpu.async_copy(
                tab_hbm.at[idx_v.at[pl.ds(w * GW, GW)]],
                buf.at[pl.ds(w * GW, GW), :],
                sem,
            )
            for w in range(NWIN)
        ]
        pltpu.sync_copy(cos_hbm.at[pl.ds(base, CB)], cos_v)
        for cp in copies:
            cp.wait()

        @pl.loop(0, GRP)
        def _grp(g):
            o = g * 16
            rows = lax.broadcasted_iota(jnp.int32, (16,), 0) + o
            cg = cos_v[pl.ds(o, 16)]
            c1v = cg * C_Y1
            c2v = cg * cg * (3.0 * C_Y2) - C_Y2
            for c in range(18):
                colv = jnp.full((16,), c, jnp.int32)
                v = plsc.load_gather(buf, [rows, colv])
                if 6 <= c < 12:
                    v = v * c1v
                elif c >= 12:
                    v = v * c2v
                plsc.store_scatter(obuf, [rows, colv], v)

        pltpu.sync_copy(obuf, out_hbm.at[pl.ds(base, CB), :])


@functools.lru_cache(maxsize=1)
def _get_sc_call():
    mesh = plsc.VectorSubcoreMesh(core_axis_name="c", subcore_axis_name="s")
    return pl.kernel(
        _sc_body,
        compiler_params=pltpu.CompilerParams(
            needs_layout_passes=False, use_tc_tiling_on_sc=False
        ),
        out_type=jax.ShapeDtypeStruct((T_TRIPLETS, 18), jnp.float32),
        mesh=mesh,
        scratch_types=[
            pltpu.VMEM((CB,), jnp.int32),
            pltpu.VMEM((CB, W_TAB), jnp.float32),
            pltpu.VMEM((CB, 18), jnp.float32),
            pltpu.VMEM((CB,), jnp.float32),
            pltpu.SemaphoreType.DMA,
        ],
    )


def kernel(dist, angle, idx_kj):
    dist_rep = jnp.repeat(dist, W_TAB).reshape(_TAB_ROWS, 128)
    table = _build_table(dist_rep).reshape(E_EDGES, W_TAB)
    cosv = _build_cos(angle)
    return _get_sc_call()(table, cosv, idx_kj)
